# baseline (speedup 1.0000x reference)
import functools

import jax
import jax.numpy as jnp
from jax import lax
from jax.experimental import pallas as pl
from jax.experimental.pallas import tpu as pltpu
from jax.experimental.pallas import tpu_sc as plsc

VOCAB = 1000000
EMBED = 16
L = 200
B = 16384

RPB = 8192
NTB = -(-VOCAB // RPB)
TLEN = NTB * RPB

NC = 2
NS = 16
NW = NC * NS
CPT = B // NW
CH = 128
NCH = CPT // CH
TSH = TLEN // NS


def _tbuild_body(emb_ref, w_ref, b_ref, out_ref):
    t = jnp.sum(emb_ref[...] * w_ref[...], axis=1) + b_ref[0, 0]

    @pl.when(pl.program_id(0) == 0)
    def _():
        r = lax.iota(jnp.int32, RPB)
        out_ref[...] = jnp.where(r == 0, b_ref[0, 0], t)

    @pl.when(pl.program_id(0) != 0)
    def _():
        out_ref[...] = t


def _tbuild(emb, w, b11):
    return pl.pallas_call(
        _tbuild_body,
        grid=(NTB,),
        in_specs=[
            pl.BlockSpec((RPB, EMBED), lambda i: (i, 0)),
            pl.BlockSpec((1, EMBED), lambda i: (0, 0)),
            pl.BlockSpec(memory_space=pltpu.SMEM),
        ],
        out_specs=pl.BlockSpec((RPB,), lambda i: (i,)),
        out_shape=jax.ShapeDtypeStruct((TLEN,), jnp.float32),
    )(emb, w, b11)


def _sc_pool(t, x):
    mesh = plsc.VectorSubcoreMesh(core_axis_name="c", subcore_axis_name="s")

    @functools.partial(
        pl.kernel,
        out_type=jax.ShapeDtypeStruct((B,), jnp.float32),
        mesh=mesh,
        scratch_types=[
            pltpu.VMEM_SHARED((TLEN,), jnp.float32),
            pltpu.VMEM((L * CH,), jnp.int32),
            pltpu.VMEM((L * CH,), jnp.float32),
            pltpu.VMEM((CH,), jnp.float32),
            pltpu.SemaphoreType.DMA,
            pltpu.SemaphoreType.DMA,
            pltpu.SemaphoreType.DMA,
        ],
    )
    def run(t_hbm, xr_hbm, out_hbm, t_sh, idx_v, val_v, o_v, tsem, isem, gsem):
        cid = lax.axis_index("c")
        sid = lax.axis_index("s")
        wid = sid * NC + cid
        base = wid * CPT

        tcopy = pltpu.async_copy(
            t_hbm.at[pl.ds(sid * TSH, TSH)],
            t_sh.at[pl.ds(sid * TSH, TSH)],
            tsem,
        )

        def stage(i):
            return pltpu.async_copy(
                xr_hbm.at[pl.ds((wid * NCH + i) * L * CH, L * CH)],
                idx_v,
                isem,
            )

        pending = stage(0)
        tcopy.wait()
        plsc.subcore_barrier()

        for i in range(NCH):
            pending.wait()
            gcopy = pltpu.async_copy(t_sh.at[idx_v], val_v, gsem)
            gcopy.wait()
            pending = stage(i + 1) if i + 1 < NCH else None

            def lbody(l, accs):
                return tuple(
                    accs[k] + val_v[pl.ds(l * CH + k * 16, 16)]
                    for k in range(CH // 16)
                )

            accs = lax.fori_loop(
                0, L, lbody,
                tuple(jnp.zeros((16,), jnp.float32) for _ in range(CH // 16)),
            )
            for k in range(CH // 16):
                z = accs[k] * (1.0 / L)
                o_v[pl.ds(k * 16, 16)] = 1.0 / (1.0 + jnp.exp(-z))
            pltpu.sync_copy(o_v, out_hbm.at[pl.ds(base + i * CH, CH)])

    return run(t, x)


def kernel(x, emb_table, fc_w, fc_b):
    t = _tbuild(
        emb_table.astype(jnp.float32),
        fc_w.astype(jnp.float32),
        fc_b.reshape(1, 1).astype(jnp.float32),
    )
    xr = (
        x.astype(jnp.int32)
        .reshape(L, NW * NCH, CH)
        .transpose(1, 0, 2)
        .reshape(-1)
    )
    return _sc_pool(t, xr)


# speedup vs baseline: 1.3481x; 1.3481x over previous
import functools

import jax
import jax.numpy as jnp
from jax import lax
from jax.experimental import pallas as pl
from jax.experimental.pallas import tpu as pltpu
from jax.experimental.pallas import tpu_sc as plsc

VOCAB = 1000000
EMBED = 16
L = 200
B = 16384

GPR = 128 // EMBED
EROW = VOCAB // GPR
RPB = 1024
NTB = -(-EROW // RPB)
TLEN = NTB * RPB * GPR

NC = 2
NS = 16
NW = NC * NS
CPT = B // NW
CH = 128
NCH = CPT // CH
NCHUNK = NW * NCH
TSH = TLEN // NS


def _tbuild_body(emb_ref, w_ref, b_ref, out_ref):
    t = (
        jnp.dot(emb_ref[...], w_ref[...], preferred_element_type=jnp.float32)
        + b_ref[0, 0]
    )

    @pl.when(pl.program_id(0) == 0)
    def _():
        r = lax.broadcasted_iota(jnp.int32, (RPB, GPR), 0)
        c = lax.broadcasted_iota(jnp.int32, (RPB, GPR), 1)
        out_ref[...] = jnp.where((r == 0) & (c == 0), b_ref[0, 0], t)

    @pl.when(pl.program_id(0) != 0)
    def _():
        out_ref[...] = t


def _tbuild(emb128, wbig, b11):
    return pl.pallas_call(
        _tbuild_body,
        grid=(NTB,),
        in_specs=[
            pl.BlockSpec((RPB, 128), lambda i: (i, 0)),
            pl.BlockSpec((128, GPR), lambda i: (0, 0)),
            pl.BlockSpec(memory_space=pltpu.SMEM),
        ],
        out_specs=pl.BlockSpec((RPB, GPR), lambda i: (i, 0)),
        out_shape=jax.ShapeDtypeStruct((NTB * RPB, GPR), jnp.float32),
    )(emb128, wbig, b11)


def _relayout_body(x_ref, out_ref):
    out_ref[0] = x_ref[...]


def _relayout(x):
    return pl.pallas_call(
        _relayout_body,
        grid=(NCHUNK,),
        in_specs=[pl.BlockSpec((L, CH), lambda i: (0, i))],
        out_specs=pl.BlockSpec((1, L, CH), lambda i: (i, 0, 0)),
        out_shape=jax.ShapeDtypeStruct((NCHUNK, L, CH), jnp.int32),
    )(x)


def _sc_pool(t, xr):
    mesh = plsc.VectorSubcoreMesh(core_axis_name="c", subcore_axis_name="s")

    @functools.partial(
        pl.kernel,
        out_type=jax.ShapeDtypeStruct((B,), jnp.float32),
        mesh=mesh,
        scratch_types=[
            pltpu.VMEM_SHARED((TLEN,), jnp.float32),
            pltpu.VMEM((L * CH,), jnp.int32),
            pltpu.VMEM((L * CH,), jnp.float32),
            pltpu.VMEM((CH,), jnp.float32),
            pltpu.SemaphoreType.DMA,
            pltpu.SemaphoreType.DMA,
            pltpu.SemaphoreType.DMA,
        ],
    )
    def run(t_hbm, xr_hbm, out_hbm, t_sh, idx_v, val_v, o_v, tsem, isem, gsem):
        cid = lax.axis_index("c")
        sid = lax.axis_index("s")
        wid = sid * NC + cid
        base = wid * CPT

        tcopy = pltpu.async_copy(
            t_hbm.at[pl.ds(sid * TSH, TSH)],
            t_sh.at[pl.ds(sid * TSH, TSH)],
            tsem,
        )

        def stage(i):
            return pltpu.async_copy(
                xr_hbm.at[pl.ds((wid * NCH + i) * L * CH, L * CH)],
                idx_v,
                isem,
            )

        pending = stage(0)
        tcopy.wait()
        plsc.subcore_barrier()

        for i in range(NCH):
            pending.wait()
            pltpu.async_copy(t_sh.at[idx_v], val_v, gsem).wait()
            pending = stage(i + 1) if i + 1 < NCH else None

            def lbody(l, accs):
                return tuple(
                    accs[k] + val_v[pl.ds(l * CH + k * 16, 16)]
                    for k in range(CH // 16)
                )

            accs = lax.fori_loop(
                0, L, lbody,
                tuple(jnp.zeros((16,), jnp.float32) for _ in range(CH // 16)),
            )
            for k in range(CH // 16):
                z = accs[k] * (1.0 / L)
                o_v[pl.ds(k * 16, 16)] = 1.0 / (1.0 + jnp.exp(-z))
            pltpu.sync_copy(o_v, out_hbm.at[pl.ds(base + i * CH, CH)])

    return run(t, xr)


def kernel(x, emb_table, fc_w, fc_b):
    w = fc_w.reshape(-1).astype(jnp.float32)
    r = jnp.arange(128)
    wbig = (
        jnp.zeros((128, GPR), jnp.float32)
        .at[r, r // EMBED]
        .set(w[r % EMBED])
    )
    emb128 = emb_table.astype(jnp.float32).reshape(EROW, 128)
    t = _tbuild(emb128, wbig, fc_b.reshape(1, 1).astype(jnp.float32))
    xr = _relayout(x.astype(jnp.int32))
    return _sc_pool(t.reshape(-1), xr.reshape(-1))
